# Initial kernel scaffold; baseline (speedup 1.0000x reference)
import jax
import jax.numpy as jnp
from jax.experimental import pallas as pl


def kernel(x, w1, b1, w2, b2, conn1_out, conn1_in, conn2_out, conn2_in):
    raise NotImplementedError("write your pallas kernel here")


# speedup vs baseline: 351.2213x; 351.2213x over previous
import functools

import jax
import jax.numpy as jnp
from jax import lax
from jax.experimental import pallas as pl
from jax.experimental.pallas import tpu as pltpu
from jax.experimental.pallas import tpu_sc as plsc

_BATCH = 16384
_NF = 512
_NN = 512
_NT = 1
_NNZ1 = 32768
_NNZ2 = 512

_L = 16


def _densify(conn1_in, conn1_out, w1, conn2_in, conn2_out, w2):
    info = plsc.get_sparse_core_info()
    nc, ns = info.num_cores, info.num_subcores
    nw = nc * ns
    rows1 = _NF // nw
    rows2 = _NN // nw
    mesh = plsc.VectorSubcoreMesh(core_axis_name="c", subcore_axis_name="s")

    @functools.partial(
        pl.kernel,
        out_type=(
            jax.ShapeDtypeStruct((_NF * _NN,), jnp.float32),
            jax.ShapeDtypeStruct((_NN * _NT,), jnp.float32),
        ),
        mesh=mesh,
        scratch_types=dict(
            ci_v=pltpu.VMEM((_NNZ1,), jnp.int32),
            co_v=pltpu.VMEM((_NNZ1,), jnp.int32),
            w_v=pltpu.VMEM((_NNZ1,), jnp.float32),
            ci2_v=pltpu.VMEM((_NNZ2,), jnp.int32),
            co2_v=pltpu.VMEM((_NNZ2,), jnp.int32),
            w2_v=pltpu.VMEM((_NNZ2,), jnp.float32),
            tbl1_v=pltpu.VMEM((rows1 * _NN,), jnp.float32),
            tbl2_v=pltpu.VMEM((rows2 * _NT,), jnp.float32),
        ),
        compiler_params=pltpu.CompilerParams(needs_layout_passes=False),
    )
    def k(ci1_hbm, co1_hbm, w1_hbm, ci2_hbm, co2_hbm, w2_hbm,
          w1d_hbm, w2d_hbm,
          ci_v, co_v, w_v, ci2_v, co2_v, w2_v, tbl1_v, tbl2_v):
        wid = lax.axis_index("s") * nc + lax.axis_index("c")

        pltpu.sync_copy(ci1_hbm, ci_v)
        pltpu.sync_copy(co1_hbm, co_v)
        pltpu.sync_copy(w1_hbm, w_v)
        pltpu.sync_copy(ci2_hbm, ci2_v)
        pltpu.sync_copy(co2_hbm, co2_v)
        pltpu.sync_copy(w2_hbm, w2_v)

        zero = jnp.zeros((_L,), jnp.float32)

        def zero1(i, _):
            tbl1_v[pl.ds(i * _L, _L)] = zero
            return 0
        lax.fori_loop(0, rows1 * _NN // _L, zero1, 0, unroll=8)
        tbl2_v[pl.ds(0, _L)] = zero

        base1 = wid * rows1

        def scat1(i, _):
            ci = ci_v[pl.ds(i * _L, _L)]
            co = co_v[pl.ds(i * _L, _L)]
            wv = w_v[pl.ds(i * _L, _L)]
            r = ci - base1
            m = (r >= 0) & (r < rows1)
            loc = jnp.where(m, r * _NN + co, 0)
            val = jnp.where(m, wv, 0.0)
            plsc.addupdate_scatter(tbl1_v, [loc], val, mask=m)
            return 0
        lax.fori_loop(0, _NNZ1 // _L, scat1, 0, unroll=4)

        base2 = wid * rows2

        def scat2(i, _):
            ci = ci2_v[pl.ds(i * _L, _L)]
            co = co2_v[pl.ds(i * _L, _L)]
            wv = w2_v[pl.ds(i * _L, _L)]
            r = ci - base2
            m = (r >= 0) & (r < rows2)
            loc = jnp.where(m, r * _NT + co, 0)
            val = jnp.where(m, wv, 0.0)
            plsc.addupdate_scatter(tbl2_v, [loc], val, mask=m)
            return 0
        lax.fori_loop(0, _NNZ2 // _L, scat2, 0, unroll=4)

        pltpu.sync_copy(tbl1_v, w1d_hbm.at[pl.ds(base1 * _NN, rows1 * _NN)])
        pltpu.sync_copy(tbl2_v, w2d_hbm.at[pl.ds(base2 * _NT, rows2 * _NT)])

    return k(conn1_in, conn1_out, w1, conn2_in, conn2_out, w2)


_BLK = 1024


def _mlp_body(x_ref, w1_ref, b1_ref, w2_ref, b2_ref, o_ref):
    h = jnp.dot(x_ref[...], w1_ref[...], preferred_element_type=jnp.float32)
    h = jax.nn.sigmoid(h + b1_ref[...])
    y = jnp.dot(h, w2_ref[...], preferred_element_type=jnp.float32)
    o_ref[...] = jax.nn.sigmoid(y + b2_ref[...])


def _mlp(x, w1d, b1, w2d, b2):
    grid = (_BATCH // _BLK,)
    return pl.pallas_call(
        _mlp_body,
        grid=grid,
        in_specs=[
            pl.BlockSpec((_BLK, _NF), lambda i: (i, 0)),
            pl.BlockSpec((_NF, _NN), lambda i: (0, 0)),
            pl.BlockSpec((1, _NN), lambda i: (0, 0)),
            pl.BlockSpec((_NN, _NT), lambda i: (0, 0)),
            pl.BlockSpec((1, _NT), lambda i: (0, 0)),
        ],
        out_specs=pl.BlockSpec((_BLK, _NT), lambda i: (i, 0)),
        out_shape=jax.ShapeDtypeStruct((_BATCH, _NT), jnp.float32),
    )(x, w1d, b1, w2d, b2)


def kernel(x, w1, b1, w2, b2, conn1_out, conn1_in, conn2_out, conn2_in):
    w1d_flat, w2d_flat = _densify(conn1_in, conn1_out, w1,
                                  conn2_in, conn2_out, w2)
    w1d = w1d_flat.reshape(_NF, _NN)
    w2d = w2d_flat.reshape(_NN, _NT)
    return _mlp(x, w1d, b1.reshape(1, _NN), w2d, b2.reshape(1, _NT))


# speedup vs baseline: 416.6820x; 1.1864x over previous
import functools

import jax
import jax.numpy as jnp
from jax import lax
from jax.experimental import pallas as pl
from jax.experimental.pallas import tpu as pltpu
from jax.experimental.pallas import tpu_sc as plsc

_BATCH = 16384
_NF = 512
_NN = 512
_NT = 1
_NNZ1 = 32768
_NNZ2 = 512

_L = 16


def _densify(conn1_in, conn1_out, w1, conn2_in, conn2_out, w2):
    info = plsc.get_sparse_core_info()
    nc, ns = info.num_cores, info.num_subcores
    nw = nc * ns
    rows1 = _NF // nw
    rows2 = _NN // nw
    mesh = plsc.VectorSubcoreMesh(core_axis_name="c", subcore_axis_name="s")

    @functools.partial(
        pl.kernel,
        out_type=(
            jax.ShapeDtypeStruct((_NF * _NN,), jnp.float32),
            jax.ShapeDtypeStruct((_NN * _NT,), jnp.float32),
        ),
        mesh=mesh,
        scratch_types=dict(
            ci_v=pltpu.VMEM((_NNZ1,), jnp.int32),
            co_v=pltpu.VMEM((_NNZ1,), jnp.int32),
            w_v=pltpu.VMEM((_NNZ1,), jnp.float32),
            ci2_v=pltpu.VMEM((_NNZ2,), jnp.int32),
            co2_v=pltpu.VMEM((_NNZ2,), jnp.int32),
            w2_v=pltpu.VMEM((_NNZ2,), jnp.float32),
            tbl1_v=pltpu.VMEM((rows1 * _NN,), jnp.float32),
            tbl2_v=pltpu.VMEM((rows2 * _NT,), jnp.float32),
            sem=pltpu.SemaphoreType.DMA,
        ),
        compiler_params=pltpu.CompilerParams(needs_layout_passes=False),
    )
    def k(ci1_hbm, co1_hbm, w1_hbm, ci2_hbm, co2_hbm, w2_hbm,
          w1d_hbm, w2d_hbm,
          ci_v, co_v, w_v, ci2_v, co2_v, w2_v, tbl1_v, tbl2_v, sem):
        wid = lax.axis_index("s") * nc + lax.axis_index("c")

        cps = [
            pltpu.async_copy(ci1_hbm, ci_v, sem),
            pltpu.async_copy(co1_hbm, co_v, sem),
            pltpu.async_copy(w1_hbm, w_v, sem),
            pltpu.async_copy(ci2_hbm, ci2_v, sem),
            pltpu.async_copy(co2_hbm, co2_v, sem),
            pltpu.async_copy(w2_hbm, w2_v, sem),
        ]

        zero = jnp.zeros((_L,), jnp.float32)

        @plsc.parallel_loop(0, rows1 * _NN // _L, unroll=8)
        def _(i):
            tbl1_v[pl.ds(i * _L, _L)] = zero
        tbl2_v[pl.ds(0, _L)] = zero

        for cp in cps:
            cp.wait()

        base1 = wid * rows1

        @plsc.parallel_loop(0, _NNZ1 // _L, unroll=8)
        def _(i):
            ci = ci_v[pl.ds(i * _L, _L)]
            co = co_v[pl.ds(i * _L, _L)]
            wv = w_v[pl.ds(i * _L, _L)]
            r = ci - base1
            m = (r >= 0) & (r < rows1)
            loc = jnp.where(m, r * _NN + co, 0)
            val = jnp.where(m, wv, 0.0)
            plsc.addupdate_scatter(tbl1_v, [loc], val, mask=m)

        base2 = wid * rows2

        @plsc.parallel_loop(0, _NNZ2 // _L, unroll=4)
        def _(i):
            ci = ci2_v[pl.ds(i * _L, _L)]
            co = co2_v[pl.ds(i * _L, _L)]
            wv = w2_v[pl.ds(i * _L, _L)]
            r = ci - base2
            m = (r >= 0) & (r < rows2)
            loc = jnp.where(m, r * _NT + co, 0)
            val = jnp.where(m, wv, 0.0)
            plsc.addupdate_scatter(tbl2_v, [loc], val, mask=m)

        pltpu.sync_copy(tbl1_v, w1d_hbm.at[pl.ds(base1 * _NN, rows1 * _NN)])
        pltpu.sync_copy(tbl2_v, w2d_hbm.at[pl.ds(base2 * _NT, rows2 * _NT)])

    return k(conn1_in, conn1_out, w1, conn2_in, conn2_out, w2)


_BLK = 1024


def _sigmoid(z):
    return 0.5 * jnp.tanh(0.5 * z) + 0.5


def _mlp_body(x_ref, w1_ref, b1_ref, w2_ref, b2_ref, o_ref):
    xb = x_ref[...].astype(jnp.bfloat16)
    w1b = w1_ref[...].astype(jnp.bfloat16)
    h = jnp.dot(xb, w1b, preferred_element_type=jnp.float32)
    h = _sigmoid(h + b1_ref[...])
    w2b = w2_ref[...].astype(jnp.bfloat16)
    y = jnp.dot(h.astype(jnp.bfloat16), w2b, preferred_element_type=jnp.float32)
    o_ref[...] = _sigmoid(y + b2_ref[...])


def _mlp(x, w1d, b1, w2d, b2):
    grid = (_BATCH // _BLK,)
    return pl.pallas_call(
        _mlp_body,
        grid=grid,
        in_specs=[
            pl.BlockSpec((_BLK, _NF), lambda i: (i, 0)),
            pl.BlockSpec((_NF, _NN), lambda i: (0, 0)),
            pl.BlockSpec((1, _NN), lambda i: (0, 0)),
            pl.BlockSpec((_NN, _NT), lambda i: (0, 0)),
            pl.BlockSpec((1, _NT), lambda i: (0, 0)),
        ],
        out_specs=pl.BlockSpec((_BLK, _NT), lambda i: (i, 0)),
        out_shape=jax.ShapeDtypeStruct((_BATCH, _NT), jnp.float32),
    )(x, w1d, b1, w2d, b2)


def kernel(x, w1, b1, w2, b2, conn1_out, conn1_in, conn2_out, conn2_in):
    w1d_flat, w2d_flat = _densify(conn1_in, conn1_out, w1,
                                  conn2_in, conn2_out, w2)
    w1d = w1d_flat.reshape(_NF, _NN)
    w2d = w2d_flat.reshape(_NN, _NT)
    return _mlp(x, w1d, b1.reshape(1, _NN), w2d, b2.reshape(1, _NT))


# speedup vs baseline: 496.7693x; 1.1922x over previous
import functools

import jax
import jax.numpy as jnp
from jax import lax
from jax.experimental import pallas as pl
from jax.experimental.pallas import tpu as pltpu
from jax.experimental.pallas import tpu_sc as plsc

_BATCH = 16384
_NF = 512
_NN = 512
_NT = 1
_NNZ1 = 32768
_NNZ2 = 512

_L = 16


def _densify(conn1_in, conn1_out, w1, conn2_in, conn2_out, w2):
    info = plsc.get_sparse_core_info()
    nc, ns = info.num_cores, info.num_subcores
    half = _NNZ1 // nc
    rows1 = _NF // ns
    rows2 = _NN // ns
    mesh = plsc.VectorSubcoreMesh(core_axis_name="c", subcore_axis_name="s")

    @functools.partial(
        pl.kernel,
        out_type=(
            jax.ShapeDtypeStruct((_NF, _NN), jnp.float32),
            jax.ShapeDtypeStruct((_NF, _NN), jnp.float32),
            jax.ShapeDtypeStruct((_NN * _NT,), jnp.float32),
        ),
        mesh=mesh,
        scratch_types=dict(
            ci_v=pltpu.VMEM((half,), jnp.int32),
            co_v=pltpu.VMEM((half,), jnp.int32),
            w_v=pltpu.VMEM((half,), jnp.float32),
            ci2_v=pltpu.VMEM((_NNZ2,), jnp.int32),
            co2_v=pltpu.VMEM((_NNZ2,), jnp.int32),
            w2_v=pltpu.VMEM((_NNZ2,), jnp.float32),
            tbl1_v=pltpu.VMEM((rows1, _NN), jnp.float32),
            tbl2_v=pltpu.VMEM((rows2 * _NT,), jnp.float32),
            sem=pltpu.SemaphoreType.DMA,
        ),
        compiler_params=pltpu.CompilerParams(needs_layout_passes=False),
    )
    def k(ci1_hbm, co1_hbm, w1_hbm, ci2_hbm, co2_hbm, w2_hbm,
          w1a_hbm, w1b_hbm, w2d_hbm,
          ci_v, co_v, w_v, ci2_v, co2_v, w2_v, tbl1_v, tbl2_v, sem):
        cid = lax.axis_index("c")
        sid = lax.axis_index("s")
        ebase = cid * half

        cps = [
            pltpu.async_copy(ci1_hbm.at[pl.ds(ebase, half)], ci_v, sem),
            pltpu.async_copy(co1_hbm.at[pl.ds(ebase, half)], co_v, sem),
            pltpu.async_copy(w1_hbm.at[pl.ds(ebase, half)], w_v, sem),
            pltpu.async_copy(ci2_hbm, ci2_v, sem),
            pltpu.async_copy(co2_hbm, co2_v, sem),
            pltpu.async_copy(w2_hbm, w2_v, sem),
        ]

        zero = jnp.zeros((_L,), jnp.float32)
        nchunk = _NN // _L

        @plsc.parallel_loop(0, rows1 * nchunk, unroll=8)
        def _(i):
            tbl1_v[i // nchunk, pl.ds((i % nchunk) * _L, _L)] = zero

        @plsc.parallel_loop(0, rows2 * _NT // _L, unroll=2)
        def _(i):
            tbl2_v[pl.ds(i * _L, _L)] = zero

        for cp in cps:
            cp.wait()

        base1 = sid * rows1

        @plsc.parallel_loop(0, half // _L, unroll=8)
        def _(i):
            ci = ci_v[pl.ds(i * _L, _L)]
            co = co_v[pl.ds(i * _L, _L)]
            wv = w_v[pl.ds(i * _L, _L)]
            r = ci - base1
            m = (r >= 0) & (r < rows1)
            rr = jnp.where(m, r, 0)
            val = jnp.where(m, wv, 0.0)
            plsc.addupdate_scatter(tbl1_v, [rr, co], val, mask=m)

        base2 = sid * rows2

        @pl.when(cid == 0)
        def _():
            @plsc.parallel_loop(0, _NNZ2 // _L, unroll=4)
            def _(i):
                ci = ci2_v[pl.ds(i * _L, _L)]
                co = co2_v[pl.ds(i * _L, _L)]
                wv = w2_v[pl.ds(i * _L, _L)]
                r = ci - base2
                m = (r >= 0) & (r < rows2)
                loc = jnp.where(m, r * _NT + co, 0)
                val = jnp.where(m, wv, 0.0)
                plsc.addupdate_scatter(tbl2_v, [loc], val, mask=m)

        @pl.when(cid == 0)
        def _():
            pltpu.sync_copy(tbl1_v, w1a_hbm.at[pl.ds(base1, rows1), :])
            pltpu.sync_copy(tbl2_v, w2d_hbm.at[pl.ds(base2 * _NT, rows2 * _NT)])

        @pl.when(cid == 1)
        def _():
            pltpu.sync_copy(tbl1_v, w1b_hbm.at[pl.ds(base1, rows1), :])

    return k(conn1_in, conn1_out, w1, conn2_in, conn2_out, w2)


_BLK = 1024


def _sigmoid(z):
    return 0.5 * jnp.tanh(0.5 * z) + 0.5


def _mlp_body(x_ref, w1a_ref, w1b_ref, b1_ref, w2_ref, b2_ref, o_ref):
    xb = x_ref[...].astype(jnp.bfloat16)
    w1b16 = (w1a_ref[...] + w1b_ref[...]).astype(jnp.bfloat16)
    h = jnp.dot(xb, w1b16, preferred_element_type=jnp.float32)
    h = _sigmoid(h + b1_ref[...])
    y = jnp.sum(h * w2_ref[...], axis=1, keepdims=True)
    o_ref[...] = _sigmoid(y + b2_ref[...])


def _mlp(x, w1a, w1b, b1, w2d, b2):
    grid = (_BATCH // _BLK,)
    return pl.pallas_call(
        _mlp_body,
        grid=grid,
        in_specs=[
            pl.BlockSpec((_BLK, _NF), lambda i: (i, 0)),
            pl.BlockSpec((_NF, _NN), lambda i: (0, 0)),
            pl.BlockSpec((_NF, _NN), lambda i: (0, 0)),
            pl.BlockSpec((_NN,), lambda i: (0,)),
            pl.BlockSpec((_NN * _NT,), lambda i: (0,)),
            pl.BlockSpec((_NT,), lambda i: (0,)),
        ],
        out_specs=pl.BlockSpec((_BLK, _NT), lambda i: (i, 0)),
        out_shape=jax.ShapeDtypeStruct((_BATCH, _NT), jnp.float32),
    )(x, w1a, w1b, b1, w2d, b2)


def kernel(x, w1, b1, w2, b2, conn1_out, conn1_in, conn2_out, conn2_in):
    w1a, w1b, w2d = _densify(conn1_in, conn1_out, w1,
                             conn2_in, conn2_out, w2)
    return _mlp(x, w1a, w1b, b1, w2d, b2)


# speedup vs baseline: 575.1254x; 1.1577x over previous
import functools

import jax
import jax.numpy as jnp
from jax import lax
from jax.experimental import pallas as pl
from jax.experimental.pallas import tpu as pltpu
from jax.experimental.pallas import tpu_sc as plsc

_BATCH = 16384
_NF = 512
_NN = 512
_NT = 1
_NNZ1 = 32768
_NNZ2 = 512

_L = 16


def _densify(conn1_in, conn1_out, w1, conn2_in, conn2_out, w2):
    info = plsc.get_sparse_core_info()
    nc, ns = info.num_cores, info.num_subcores
    half = _NNZ1 // nc
    rows1 = _NF // ns
    rows2 = _NN // ns
    mesh = plsc.VectorSubcoreMesh(core_axis_name="c", subcore_axis_name="s")

    @functools.partial(
        pl.kernel,
        out_type=(
            jax.ShapeDtypeStruct((_NF, _NN), jnp.float32),
            jax.ShapeDtypeStruct((_NF, _NN), jnp.float32),
            jax.ShapeDtypeStruct((_NN * _NT,), jnp.float32),
        ),
        mesh=mesh,
        scratch_types=dict(
            ci_v=pltpu.VMEM((half,), jnp.int32),
            co_v=pltpu.VMEM((half,), jnp.int32),
            w_v=pltpu.VMEM((half,), jnp.float32),
            ci2_v=pltpu.VMEM((_NNZ2,), jnp.int32),
            co2_v=pltpu.VMEM((_NNZ2,), jnp.int32),
            w2_v=pltpu.VMEM((_NNZ2,), jnp.float32),
            tbl1_v=pltpu.VMEM((rows1, _NN), jnp.float32),
            tbl2_v=pltpu.VMEM((rows2 * _NT,), jnp.float32),
            sem=pltpu.SemaphoreType.DMA,
        ),
        compiler_params=pltpu.CompilerParams(needs_layout_passes=False),
    )
    def k(ci1_hbm, co1_hbm, w1_hbm, ci2_hbm, co2_hbm, w2_hbm,
          w1a_hbm, w1b_hbm, w2d_hbm,
          ci_v, co_v, w_v, ci2_v, co2_v, w2_v, tbl1_v, tbl2_v, sem):
        cid = lax.axis_index("c")
        sid = lax.axis_index("s")
        ebase = cid * half

        cps = [
            pltpu.async_copy(ci1_hbm.at[pl.ds(ebase, half)], ci_v, sem),
            pltpu.async_copy(co1_hbm.at[pl.ds(ebase, half)], co_v, sem),
            pltpu.async_copy(w1_hbm.at[pl.ds(ebase, half)], w_v, sem),
            pltpu.async_copy(ci2_hbm, ci2_v, sem),
            pltpu.async_copy(co2_hbm, co2_v, sem),
            pltpu.async_copy(w2_hbm, w2_v, sem),
        ]

        zero = jnp.zeros((_L,), jnp.float32)
        nchunk = _NN // _L

        @plsc.parallel_loop(0, rows1 * nchunk, unroll=8)
        def _(i):
            tbl1_v[i // nchunk, pl.ds((i % nchunk) * _L, _L)] = zero

        @plsc.parallel_loop(0, rows2 * _NT // _L, unroll=2)
        def _(i):
            tbl2_v[pl.ds(i * _L, _L)] = zero

        for cp in cps:
            cp.wait()

        base1 = sid * rows1

        @plsc.parallel_loop(0, half // _L, unroll=8)
        def _(i):
            ci = ci_v[pl.ds(i * _L, _L)]
            co = co_v[pl.ds(i * _L, _L)]
            wv = w_v[pl.ds(i * _L, _L)]
            r = ci - base1
            m = (r >= 0) & (r < rows1)
            rr = jnp.where(m, r, 0)
            val = jnp.where(m, wv, 0.0)
            plsc.addupdate_scatter(tbl1_v, [rr, co], val, mask=m)

        base2 = sid * rows2

        @pl.when(cid == 0)
        def _():
            @plsc.parallel_loop(0, _NNZ2 // _L, unroll=4)
            def _(i):
                ci = ci2_v[pl.ds(i * _L, _L)]
                co = co2_v[pl.ds(i * _L, _L)]
                wv = w2_v[pl.ds(i * _L, _L)]
                r = ci - base2
                m = (r >= 0) & (r < rows2)
                loc = jnp.where(m, r * _NT + co, 0)
                val = jnp.where(m, wv, 0.0)
                plsc.addupdate_scatter(tbl2_v, [loc], val, mask=m)

        @pl.when(cid == 0)
        def _():
            pltpu.sync_copy(tbl1_v, w1a_hbm.at[pl.ds(base1, rows1), :])
            pltpu.sync_copy(tbl2_v, w2d_hbm.at[pl.ds(base2 * _NT, rows2 * _NT)])

        @pl.when(cid == 1)
        def _():
            pltpu.sync_copy(tbl1_v, w1b_hbm.at[pl.ds(base1, rows1), :])

    return k(conn1_in, conn1_out, w1, conn2_in, conn2_out, w2)


_BLK = 1024


def _sigmoid(z):
    return 0.5 * jnp.tanh(0.5 * z) + 0.5


def _mlp_body(x_ref, w1a_ref, w1b_ref, b1_ref, w2_ref, b2_ref, o_ref):
    xb = x_ref[...].astype(jnp.bfloat16)
    w1b16 = (w1a_ref[...] + w1b_ref[...]).astype(jnp.bfloat16)
    h = jnp.dot(xb, w1b16, preferred_element_type=jnp.float32)
    h = _sigmoid(h + b1_ref[...])
    w2row = w2_ref[...].reshape(1, _NN).astype(jnp.bfloat16)
    y = lax.dot_general(w2row, h.astype(jnp.bfloat16),
                        (((1,), (1,)), ((), ())),
                        preferred_element_type=jnp.float32)
    o_ref[...] = _sigmoid(y + b2_ref[0])


def _mlp(x, w1a, w1b, b1, w2d, b2):
    grid = (_BATCH // _BLK,)
    return pl.pallas_call(
        _mlp_body,
        grid=grid,
        in_specs=[
            pl.BlockSpec((_BLK, _NF), lambda i: (i, 0)),
            pl.BlockSpec((_NF, _NN), lambda i: (0, 0)),
            pl.BlockSpec((_NF, _NN), lambda i: (0, 0)),
            pl.BlockSpec((_NN,), lambda i: (0,)),
            pl.BlockSpec((_NN * _NT,), lambda i: (0,)),
            pl.BlockSpec((_NT,), lambda i: (0,)),
        ],
        out_specs=pl.BlockSpec((1, _BLK), lambda i: (0, i)),
        out_shape=jax.ShapeDtypeStruct((1, _BATCH), jnp.float32),
    )(x, w1a, w1b, b1, w2d, b2)


def kernel(x, w1, b1, w2, b2, conn1_out, conn1_in, conn2_out, conn2_in):
    w1a, w1b, w2d = _densify(conn1_in, conn1_out, w1,
                             conn2_in, conn2_out, w2)
    return _mlp(x, w1a, w1b, b1, w2d, b2).reshape(_BATCH, _NT)


# speedup vs baseline: 614.5103x; 1.0685x over previous
import functools

import jax
import jax.numpy as jnp
from jax import lax
from jax.experimental import pallas as pl
from jax.experimental.pallas import tpu as pltpu
from jax.experimental.pallas import tpu_sc as plsc

_BATCH = 16384
_NF = 512
_NN = 512
_NT = 1
_NNZ1 = 32768
_NNZ2 = 512

_L = 16


def _densify(conn1_in, conn1_out, w1, conn2_in, conn2_out, w2):
    info = plsc.get_sparse_core_info()
    nc, ns = info.num_cores, info.num_subcores
    half = _NNZ1 // nc
    rows1 = _NF // ns
    rows2 = _NN // ns
    mesh = plsc.VectorSubcoreMesh(core_axis_name="c", subcore_axis_name="s")

    @functools.partial(
        pl.kernel,
        out_type=(
            jax.ShapeDtypeStruct((_NF, _NN), jnp.float32),
            jax.ShapeDtypeStruct((_NF, _NN), jnp.float32),
            jax.ShapeDtypeStruct((_NN * _NT,), jnp.float32),
        ),
        mesh=mesh,
        scratch_types=dict(
            ci_v=pltpu.VMEM((half,), jnp.int32),
            co_v=pltpu.VMEM((half,), jnp.int32),
            w_v=pltpu.VMEM((half,), jnp.float32),
            ci2_v=pltpu.VMEM((_NNZ2,), jnp.int32),
            co2_v=pltpu.VMEM((_NNZ2,), jnp.int32),
            w2_v=pltpu.VMEM((_NNZ2,), jnp.float32),
            tbl1_v=pltpu.VMEM((rows1, _NN), jnp.float32),
            tbl2_v=pltpu.VMEM((rows2 * _NT,), jnp.float32),
            sem=pltpu.SemaphoreType.DMA,
        ),
        compiler_params=pltpu.CompilerParams(needs_layout_passes=False),
    )
    def k(ci1_hbm, co1_hbm, w1_hbm, ci2_hbm, co2_hbm, w2_hbm,
          w1a_hbm, w1b_hbm, w2d_hbm,
          ci_v, co_v, w_v, ci2_v, co2_v, w2_v, tbl1_v, tbl2_v, sem):
        cid = lax.axis_index("c")
        sid = lax.axis_index("s")
        ebase = cid * half

        cps = [
            pltpu.async_copy(ci1_hbm.at[pl.ds(ebase, half)], ci_v, sem),
            pltpu.async_copy(co1_hbm.at[pl.ds(ebase, half)], co_v, sem),
            pltpu.async_copy(w1_hbm.at[pl.ds(ebase, half)], w_v, sem),
            pltpu.async_copy(ci2_hbm, ci2_v, sem),
            pltpu.async_copy(co2_hbm, co2_v, sem),
            pltpu.async_copy(w2_hbm, w2_v, sem),
        ]

        zero = jnp.zeros((_L,), jnp.float32)
        nchunk = _NN // _L

        @plsc.parallel_loop(0, rows1 * nchunk, unroll=8)
        def _(i):
            tbl1_v[i // nchunk, pl.ds((i % nchunk) * _L, _L)] = zero

        @plsc.parallel_loop(0, rows2 * _NT // _L, unroll=2)
        def _(i):
            tbl2_v[pl.ds(i * _L, _L)] = zero

        for cp in cps:
            cp.wait()

        base1 = sid * rows1

        @plsc.parallel_loop(0, half // _L, unroll=8)
        def _(i):
            ci = ci_v[pl.ds(i * _L, _L)]
            co = co_v[pl.ds(i * _L, _L)]
            wv = w_v[pl.ds(i * _L, _L)]
            r = ci - base1
            m = (r >= 0) & (r < rows1)
            rr = jnp.where(m, r, 0)
            val = jnp.where(m, wv, 0.0)
            plsc.addupdate_scatter(tbl1_v, [rr, co], val, mask=m)

        base2 = sid * rows2

        @pl.when(cid == 0)
        def _():
            @plsc.parallel_loop(0, _NNZ2 // _L, unroll=4)
            def _(i):
                ci = ci2_v[pl.ds(i * _L, _L)]
                co = co2_v[pl.ds(i * _L, _L)]
                wv = w2_v[pl.ds(i * _L, _L)]
                r = ci - base2
                m = (r >= 0) & (r < rows2)
                loc = jnp.where(m, r * _NT + co, 0)
                val = jnp.where(m, wv, 0.0)
                plsc.addupdate_scatter(tbl2_v, [loc], val, mask=m)

        @pl.when(cid == 0)
        def _():
            pltpu.sync_copy(tbl1_v, w1a_hbm.at[pl.ds(base1, rows1), :])
            pltpu.sync_copy(tbl2_v, w2d_hbm.at[pl.ds(base2 * _NT, rows2 * _NT)])

        @pl.when(cid == 1)
        def _():
            pltpu.sync_copy(tbl1_v, w1b_hbm.at[pl.ds(base1, rows1), :])

    return k(conn1_in, conn1_out, w1, conn2_in, conn2_out, w2)


_BLK = 2048


def _sigmoid(z):
    return 0.5 * jnp.tanh(0.5 * z) + 0.5


def _mlp_body(x_ref, w1a_ref, w1b_ref, b1_ref, w2_ref, b2_ref, o_ref):
    xb = x_ref[...].astype(jnp.bfloat16)
    w1b16 = (w1a_ref[...] + w1b_ref[...]).astype(jnp.bfloat16)
    h = jnp.dot(xb, w1b16, preferred_element_type=jnp.float32)
    h = _sigmoid(h + b1_ref[...])
    w2row = w2_ref[...].reshape(1, _NN).astype(jnp.bfloat16)
    y = lax.dot_general(w2row, h.astype(jnp.bfloat16),
                        (((1,), (1,)), ((), ())),
                        preferred_element_type=jnp.float32)
    o_ref[...] = _sigmoid(y + b2_ref[0])


def _mlp(x, w1a, w1b, b1, w2d, b2):
    grid = (_BATCH // _BLK,)
    return pl.pallas_call(
        _mlp_body,
        grid=grid,
        in_specs=[
            pl.BlockSpec((_BLK, _NF), lambda i: (i, 0)),
            pl.BlockSpec((_NF, _NN), lambda i: (0, 0)),
            pl.BlockSpec((_NF, _NN), lambda i: (0, 0)),
            pl.BlockSpec((_NN,), lambda i: (0,)),
            pl.BlockSpec((_NN * _NT,), lambda i: (0,)),
            pl.BlockSpec((_NT,), lambda i: (0,)),
        ],
        out_specs=pl.BlockSpec((1, _BLK), lambda i: (0, i)),
        out_shape=jax.ShapeDtypeStruct((1, _BATCH), jnp.float32),
    )(x, w1a, w1b, b1, w2d, b2)


def kernel(x, w1, b1, w2, b2, conn1_out, conn1_in, conn2_out, conn2_in):
    w1a, w1b, w2d = _densify(conn1_in, conn1_out, w1,
                             conn2_in, conn2_out, w2)
    return _mlp(x, w1a, w1b, b1, w2d, b2).reshape(_BATCH, _NT)
